# baseline (speedup 1.0000x reference)
import functools

import jax
import jax.numpy as jnp
from jax import lax
from jax.experimental import pallas as pl
from jax.experimental.pallas import tpu as pltpu
from jax.experimental.pallas import tpu_sc as plsc

_N = 1024
_E = 2048
_BLK = 256
_NB = _E // _BLK

_NC, _NS, _L = 2, 16, 16
_NT = _NC * _NS
_EPT = _E // _NT
_NG = _EPT // _L
_COLS = 4 * _N // _NS


def _silu(v):
    return v / (1.0 + jnp.exp(-v))


def _g_body(x_ref, W1_ref, b1_ref, g_ref):
    Wd = W1_ref[:_N, :] - W1_ref[_N:, :]
    Ws = W1_ref[_N:, :]
    z = jnp.zeros((_N, 14), jnp.float32)
    Wpad = jnp.concatenate([Wd, z, Ws, z], axis=1)
    b1p = jnp.concatenate([b1_ref[...], jnp.zeros((1, 78), jnp.float32)],
                          axis=1)
    g_ref[...] = (jnp.dot(x_ref[...], Wpad,
                          preferred_element_type=jnp.float32) + b1p)


def _g_call(x, W1, b1row):
    return pl.pallas_call(
        _g_body,
        out_shape=jax.ShapeDtypeStruct((_N, 128), jnp.float32),
    )(x, W1, b1row)


def _sc_conv_body(g_hbm, ei_hbm, w2s_hbm, b2s_hbm, w3s_hbm, b3s_hbm,
                  conv2_hbm,
                  idx_d, idx_s, gd, gs, w2v, b2v, w3v, b3v,
                  convloc, tmp, red, shared, sem):
    cid = lax.axis_index("c")
    sid = lax.axis_index("s")
    base = (cid * _NS + sid) * _EPT
    pltpu.sync_copy(ei_hbm.at[1, pl.ds(base, _EPT)], idx_d)
    pltpu.sync_copy(ei_hbm.at[0, pl.ds(base, _EPT)], idx_s)
    c1 = pltpu.async_copy(g_hbm.at[idx_d], gd, sem)
    c2 = pltpu.async_copy(g_hbm.at[idx_s], gs, sem)
    pltpu.sync_copy(w2s_hbm, w2v)
    pltpu.sync_copy(b2s_hbm, b2v)
    pltpu.sync_copy(w3s_hbm, w3v)
    pltpu.sync_copy(b3s_hbm, b3v)

    zero = jnp.zeros((_L,), jnp.float32)

    def _zero_step(i, _):
        convloc[pl.ds(i * _L, _L)] = zero
        return 0

    lax.fori_loop(0, 4 * _N // _L, _zero_step, 0)
    c1.wait()
    c2.wait()

    ii = lax.iota(jnp.int32, _L)

    def _group(gi, _):
        rows = gi * _L + ii
        dstv = idx_d[pl.ds(gi * _L, _L)]
        h1 = []
        for j in range(50):
            a = plsc.load_gather(gd, [rows, jnp.full((_L,), j, jnp.int32)])
            b = plsc.load_gather(gs, [rows, jnp.full((_L,), 64 + j,
                                                     jnp.int32)])
            h1.append(_silu(a + b))
        h2 = []
        for k in range(10):
            acc = b2v[pl.ds(k * _L, _L)]
            for j in range(50):
                acc = acc + h1[j] * w2v[pl.ds((j * 10 + k) * _L, _L)]
            h2.append(_silu(acc))
        for c in range(4):
            acc = b3v[pl.ds(c * _L, _L)]
            for k in range(10):
                acc = acc + h2[k] * w3v[pl.ds((k * 4 + c) * _L, _L)]
            h3 = _silu(acc)
            plsc.addupdate_scatter(convloc, [dstv * 4 + c], h3)
        return 0

    lax.fori_loop(0, _NG, _group, 0)

    pltpu.sync_copy(convloc, shared.at[sid])
    plsc.subcore_barrier()
    pltpu.sync_copy(shared.at[:, pl.ds(sid * _COLS, _COLS)], tmp)

    def _red_step(i, _):
        acc = tmp[0, pl.ds(i * _L, _L)]
        for r in range(1, _NS):
            acc = acc + tmp[r, pl.ds(i * _L, _L)]
        red[pl.ds(i * _L, _L)] = acc
        return 0

    lax.fori_loop(0, _COLS // _L, _red_step, 0)
    pltpu.sync_copy(red, conv2_hbm.at[cid, pl.ds(sid * _COLS, _COLS)])


def _sc_conv_call(G, edge_index, w2s, b2s, w3s, b3s):
    mesh = plsc.VectorSubcoreMesh(core_axis_name="c", subcore_axis_name="s",
                                  num_cores=_NC, num_subcores=_NS)
    f = pl.kernel(
        _sc_conv_body,
        out_type=jax.ShapeDtypeStruct((_NC, 4 * _N), jnp.float32),
        mesh=mesh,
        compiler_params=pltpu.CompilerParams(needs_layout_passes=False),
        scratch_types=[
            pltpu.VMEM((_EPT,), jnp.int32),
            pltpu.VMEM((_EPT,), jnp.int32),
            pltpu.VMEM((_EPT, 128), jnp.float32),
            pltpu.VMEM((_EPT, 128), jnp.float32),
            pltpu.VMEM((500 * _L,), jnp.float32),
            pltpu.VMEM((10 * _L,), jnp.float32),
            pltpu.VMEM((40 * _L,), jnp.float32),
            pltpu.VMEM((4 * _L,), jnp.float32),
            pltpu.VMEM((4 * _N,), jnp.float32),
            pltpu.VMEM((_NS, _COLS), jnp.float32),
            pltpu.VMEM((_COLS,), jnp.float32),
            pltpu.VMEM_SHARED((_NS, 4 * _N), jnp.float32),
            pltpu.SemaphoreType.DMA,
        ],
    )
    return f(G, edge_index, w2s, b2s, w3s, b3s)


def _head_body(conv2_ref, rem_ref, locks_ref, Wp_ref, bp_ref,
               Wv1_ref, bv1_ref, Wv2_ref, bv2_ref, Wv3_ref, bv3_ref,
               value_ref, policy_ref, p_scr, xf_scr):
    i = pl.program_id(0)

    @pl.when(i == 0)
    def _():
        xf_scr[...] = conv2_ref[0:1, :] + conv2_ref[1:2, :]

    @pl.when(i < _NB)
    def _():
        xf = xf_scr[...]
        blk = (jnp.dot(xf, Wp_ref[:4 * _N, :],
                       preferred_element_type=jnp.float32)
               + jnp.dot(locks_ref[...], Wp_ref[4 * _N:, :],
                         preferred_element_type=jnp.float32)
               + bp_ref[...])
        p_scr[:, pl.ds(pl.multiple_of(i * _BLK, _BLK), _BLK)] = blk

    @pl.when(i == _NB)
    def _():
        p = p_scr[...]
        p2 = p * p
        policy_ref[...] = p2 / jnp.sum(p2)
        xf = xf_scr[...]
        v = _silu(jnp.dot(xf, Wv1_ref[:4 * _N, :],
                          preferred_element_type=jnp.float32)
                  + jnp.dot(rem_ref[...], Wv1_ref[4 * _N:5 * _N, :],
                            preferred_element_type=jnp.float32)
                  + jnp.dot(locks_ref[...], Wv1_ref[5 * _N:, :],
                            preferred_element_type=jnp.float32)
                  + bv1_ref[...])
        v = _silu(jnp.dot(v, Wv2_ref[...],
                          preferred_element_type=jnp.float32) + bv2_ref[...])
        value_ref[...] = (jnp.dot(v, Wv3_ref[...],
                                  preferred_element_type=jnp.float32)
                          + bv3_ref[...])


def _head_call(conv2, rem, locks, Wp, bp, Wv1, bv1, Wv2, bv2, Wv3, bv3):
    full = lambda shape: pl.BlockSpec(shape, lambda i: (0, 0))
    return pl.pallas_call(
        _head_body,
        grid=(_NB + 1,),
        in_specs=[
            full((_NC, 4 * _N)),
            full((1, _N)),
            full((1, _E)),
            pl.BlockSpec((4 * _N + _E, _BLK),
                         lambda i: (0, jnp.minimum(i, _NB - 1))),
            pl.BlockSpec((1, _BLK), lambda i: (0, jnp.minimum(i, _NB - 1))),
            full((4 * _N + _N + _E, 64)),
            full((1, 64)),
            full((64, 16)),
            full((1, 16)),
            full((16, 1)),
            full((1, 1)),
        ],
        out_specs=[full((1, 1)), full((1, _E))],
        out_shape=[
            jax.ShapeDtypeStruct((1, 1), jnp.float32),
            jax.ShapeDtypeStruct((1, _E), jnp.float32),
        ],
        scratch_shapes=[pltpu.VMEM((1, _E), jnp.float32),
                        pltpu.VMEM((1, 4 * _N), jnp.float32)],
    )(conv2, rem, locks, Wp, bp, Wv1, bv1, Wv2, bv2, Wv3, bv3)


def kernel(x, remaining, locks, edge_index,
           W1, b1, W2, b2, W3, b3,
           Wv1, bv1, Wv2, bv2, Wv3, bv3,
           Wp, bp):
    G = _g_call(x, W1, b1.reshape(1, 50))
    w2s = jnp.broadcast_to(W2[:, :, None], (50, 10, _L)).reshape(500 * _L)
    b2s = jnp.broadcast_to(b2[:, None], (10, _L)).reshape(10 * _L)
    w3s = jnp.broadcast_to(W3[:, :, None], (10, 4, _L)).reshape(40 * _L)
    b3s = jnp.broadcast_to(b3[:, None], (4, _L)).reshape(4 * _L)
    conv2 = _sc_conv_call(G, edge_index, w2s, b2s, w3s, b3s)
    value2d, policy2d = _head_call(conv2, remaining.reshape(1, _N),
                                   locks.reshape(1, _E),
                                   Wp, bp.reshape(1, _E),
                                   Wv1, bv1.reshape(1, 64),
                                   Wv2, bv2.reshape(1, 16),
                                   Wv3, bv3.reshape(1, 1))
    return (value2d.reshape(1), policy2d.reshape(_E))
